# baseline (speedup 1.0000x reference)
import functools

import jax
import jax.numpy as jnp
from jax import lax
from jax.experimental import pallas as pl
from jax.experimental.pallas import tpu as pltpu
from jax.experimental.pallas import tpu_sc as plsc

N = 10000
E = 320000
D = 128
EPS = 1e-5

NC = 2
NS = 16
NW = NC * NS
EW = E // NW
C = 80
EWP = 10240
NCHUNK = EWP // C
NB = 2
NID = 8
NLA = 6
NP = 10240
RPT = NP // NS
RZ = C
NRW = RPT // RZ
WZ = 128
NWB = RPT // WZ


def _sc_agg_body(x_hbm, src_hbm, dst_hbm, out_hbm,
                 sidx_v, didx_v, rows_v, acc_sh, *sems):
    gsem = sems[:NB]
    ssem = sems[NB:2 * NB]
    isem = sems[2 * NB:]
    c = lax.axis_index("c")
    s = lax.axis_index("s")
    wid = s * NC + c
    ebase = wid * EWP

    def zfill(i, carry):
        rows_v[0, i // 8, pl.ds((i % 8) * 16, 16)] = jnp.zeros((16,), jnp.float32)
        return carry

    lax.fori_loop(0, RZ * 8, zfill, 0)
    for k in range(NRW):
        pltpu.sync_copy(rows_v.at[0, pl.ds(0, RZ)],
                        acc_sh.at[pl.ds(s * RPT + k * RZ, RZ)])

    def idxload(t, j):
        pltpu.async_copy(src_hbm.at[pl.ds(ebase + t * C, C)], sidx_v.at[j], isem[j])
        pltpu.async_copy(dst_hbm.at[pl.ds(ebase + t * C, C)], didx_v.at[j], isem[j])

    def iwait(j):
        pltpu.make_async_copy(src_hbm.at[pl.ds(0, C)], sidx_v.at[j], isem[j]).wait()
        pltpu.make_async_copy(dst_hbm.at[pl.ds(0, C)], didx_v.at[j], isem[j]).wait()

    def gather(t, j, b):
        del t
        pltpu.async_copy(x_hbm.at[sidx_v.at[j]], rows_v.at[b], gsem[b])

    def gwait(b):
        pltpu.make_async_copy(x_hbm.at[sidx_v.at[0]], rows_v.at[b], gsem[b]).wait()

    def scatter(t, j, b):
        del t
        pltpu.async_copy(rows_v.at[b], acc_sh.at[didx_v.at[j]], ssem[b], add=True)

    def swait(b):
        pltpu.make_async_copy(rows_v.at[b], acc_sh.at[didx_v.at[0]], ssem[b]).wait()

    for j in range(NLA):
        idxload(j, j)
    plsc.subcore_barrier()
    for t in range(NB):
        iwait(t % NID)
        gather(t, t % NID, t % NB)

    def step(t, u, guard_load, guard_gather):
        b = u % NB
        gwait(b)
        scatter(t, u % NID, b)
        if guard_load:
            idxload(t + NLA, (u + NLA) % NID)
        if guard_gather:
            iwait((u + NB) % NID)
        swait(b)
        if guard_gather:
            gather(t + NB, (u + NB) % NID, b)

    def grp(g, carry):
        for u in range(NID):
            step(g * NID + u, u, True, True)
        return carry

    NMAIN = ((NCHUNK - NLA) // NID) * NID
    lax.fori_loop(0, NMAIN // NID, grp, 0)
    for t in range(NMAIN, NCHUNK):
        step(t, t % NID, t + NLA < NCHUNK, t + NB < NCHUNK)
    plsc.subcore_barrier()

    for k in range(NWB):
        r = s * RPT + k * WZ
        pltpu.sync_copy(acc_sh.at[pl.ds(r, WZ)], out_hbm.at[pl.ds(c * NP + r, WZ)])


@functools.lru_cache(maxsize=None)
def _get_sc_agg():
    return pl.kernel(
        _sc_agg_body,
        mesh=plsc.VectorSubcoreMesh(core_axis_name="c", subcore_axis_name="s"),
        out_type=jax.ShapeDtypeStruct((2 * NP, D), jnp.float32),
        scratch_types=[
            pltpu.VMEM((NID, C), jnp.int32),
            pltpu.VMEM((NID, C), jnp.int32),
            pltpu.VMEM((NB, C, D), jnp.float32),
            pltpu.VMEM_SHARED((NP, D), jnp.float32),
        ] + [pltpu.SemaphoreType.DMA] * (2 * NB + NID),
    )


def _tc_layer_body(p_ref, W_ref, b_ref, g_ref, be_ref, o_ref):
    a = p_ref[:N, :] + p_ref[NP:NP + N, :]
    y = jnp.dot(a, W_ref[...], preferred_element_type=jnp.float32) + b_ref[...]
    mu = jnp.mean(y, axis=0, keepdims=True)
    d = y - mu
    var = jnp.mean(d * d, axis=0, keepdims=True)
    yn = d * lax.rsqrt(var + EPS) * g_ref[...] + be_ref[...]
    o_ref[...] = jnp.maximum(yn, 0.0)


def _tc_layer(parts, W, b, g, be):
    return pl.pallas_call(
        _tc_layer_body,
        out_shape=jax.ShapeDtypeStruct((N, D), jnp.float32),
    )(parts, W, b.reshape(1, D), g.reshape(1, D), be.reshape(1, D))


def kernel(x, edge_index, W1, b1, g1, be1, W2, b2, g2, be2):
    ei = edge_index.astype(jnp.int32)
    pad = ((0, 0), (0, EWP - EW))
    src = jnp.pad(ei[0].reshape(NW, EW), pad).reshape(-1)
    padv = jnp.broadcast_to(jnp.arange(EWP - EW, dtype=jnp.int32) + N,
                            (NW, EWP - EW))
    dst = jnp.concatenate([ei[1].reshape(NW, EW), padv], axis=1).reshape(-1)
    sc_agg = _get_sc_agg()
    p1 = sc_agg(x, src, dst)
    h1 = _tc_layer(p1, W1, b1, g1, be1)
    p2 = sc_agg(h1, src, dst)
    return _tc_layer(p2, W2, b2, g2, be2)


# speedup vs baseline: 1.2913x; 1.2913x over previous
import functools

import jax
import jax.numpy as jnp
from jax import lax
from jax.experimental import pallas as pl
from jax.experimental.pallas import tpu as pltpu
from jax.experimental.pallas import tpu_sc as plsc

N = 10000
E = 320000
D = 128
EPS = 1e-5

NC = 2
NS = 16
NW = NC * NS
EW = E // NW
C = 80
NCHUNK = EW // C
NP = 10240
RPT = NP // NS
RZ = 80
NRW = RPT // RZ
WZ = 128
NWB = RPT // WZ


def _sc_agg_body(x_hbm, src_hbm, dst_hbm, out_hbm,
                 src_v, dst_v, rows_v, acc_sh, sem):
    c = lax.axis_index("c")
    s = lax.axis_index("s")
    wid = s * NC + c

    def zfill(i, carry):
        rows_v[i // 8, pl.ds((i % 8) * 16, 16)] = jnp.zeros((16,), jnp.float32)
        return carry

    lax.fori_loop(0, RZ * 8, zfill, 0)
    for k in range(NRW):
        pltpu.sync_copy(rows_v.at[pl.ds(0, RZ)],
                        acc_sh.at[pl.ds(s * RPT + k * RZ, RZ)])
    plsc.subcore_barrier()

    base = wid * EW

    def body(t, carry):
        off = base + t * C
        pltpu.sync_copy(src_hbm.at[pl.ds(off, C)], src_v)
        pltpu.sync_copy(dst_hbm.at[pl.ds(off, C)], dst_v)
        pltpu.async_copy(x_hbm.at[src_v], rows_v, sem).wait()
        pltpu.sync_copy(rows_v, acc_sh.at[dst_v], add=True)
        return carry

    lax.fori_loop(0, NCHUNK, body, 0)
    plsc.subcore_barrier()

    for k in range(NWB):
        r = s * RPT + k * WZ
        pltpu.sync_copy(acc_sh.at[pl.ds(r, WZ)], out_hbm.at[pl.ds(c * NP + r, WZ)])


@functools.lru_cache(maxsize=None)
def _get_sc_agg():
    return pl.kernel(
        _sc_agg_body,
        mesh=plsc.VectorSubcoreMesh(core_axis_name="c", subcore_axis_name="s"),
        out_type=jax.ShapeDtypeStruct((2 * NP, D), jnp.float32),
        scratch_types=[
            pltpu.VMEM((C,), jnp.int32),
            pltpu.VMEM((C,), jnp.int32),
            pltpu.VMEM((C, D), jnp.float32),
            pltpu.VMEM_SHARED((NP, D), jnp.float32),
            pltpu.SemaphoreType.DMA,
        ],
    )


def _tc_layer_body(p_ref, W_ref, b_ref, g_ref, be_ref, o_ref):
    a = p_ref[:N, :] + p_ref[NP:NP + N, :]
    y = jnp.dot(a, W_ref[...], preferred_element_type=jnp.float32) + b_ref[...]
    mu = jnp.mean(y, axis=0, keepdims=True)
    d = y - mu
    var = jnp.mean(d * d, axis=0, keepdims=True)
    yn = d * lax.rsqrt(var + EPS) * g_ref[...] + be_ref[...]
    o_ref[...] = jnp.maximum(yn, 0.0)


def _tc_layer(parts, W, b, g, be):
    return pl.pallas_call(
        _tc_layer_body,
        out_shape=jax.ShapeDtypeStruct((N, D), jnp.float32),
    )(parts, W, b.reshape(1, D), g.reshape(1, D), be.reshape(1, D))


def kernel(x, edge_index, W1, b1, g1, be1, W2, b2, g2, be2):
    ei = edge_index.astype(jnp.int32)
    src, dst = ei[0], ei[1]
    sc_agg = _get_sc_agg()
    p1 = sc_agg(x, src, dst)
    h1 = _tc_layer(p1, W1, b1, g1, be1)
    p2 = sc_agg(h1, src, dst)
    return _tc_layer(p2, W2, b2, g2, be2)
